# baseline (speedup 1.0000x reference)
import jax
import jax.numpy as jnp
from jax import lax
from jax.experimental import pallas as pl
from jax.experimental.pallas import tpu as pltpu
from jax.experimental.pallas import tpu_sc as plsc

NUM_CORES = 2
NUM_SUBCORES = 16
LANES = 16
NW = NUM_CORES * NUM_SUBCORES
BATCH_SIZE = 16384
DIM = 64
BPW = BATCH_SIZE // NW
CHUNK = 128
NCHUNK = BPW // CHUNK


def _tec_body(h_hbm, r_hbm, t_hbm, ent_hbm, rel_hbm, out_hbm,
              hi, ri, ti, hv, rv, tv, ov, sem):
    wid = lax.axis_index("s") * NUM_CORES + lax.axis_index("c")
    base = wid * BPW

    pltpu.sync_copy(h_hbm.at[wid], hi)
    pltpu.sync_copy(r_hbm.at[wid], ri)
    pltpu.sync_copy(t_hbm.at[wid], ti)

    copies = []
    for c in range(NCHUNK):
        dst = pl.ds(c * CHUNK, CHUNK)
        copies.append(pltpu.async_copy(ent_hbm.at[hi.at[c]], hv.at[dst], sem))
        copies.append(pltpu.async_copy(ent_hbm.at[ti.at[c]], tv.at[dst], sem))
        copies.append(pltpu.async_copy(rel_hbm.at[ri.at[c]], rv.at[dst], sem))
    for cp in copies:
        cp.wait()

    def block(blk, carry):
        rows = blk * LANES + lax.iota(jnp.int32, LANES)

        def dbody(d, acc):
            dvec = jnp.full((LANES,), d, jnp.int32)
            hg = plsc.load_gather(hv, [rows, dvec])
            rg = plsc.load_gather(rv, [rows, dvec])
            tg = plsc.load_gather(tv, [rows, dvec])
            df = (hg - tg) + rg
            return acc + df * df

        acc = lax.fori_loop(0, DIM, dbody, jnp.zeros((LANES,), jnp.float32))

        bits = plsc.bitcast(acc, jnp.int32)
        y = plsc.bitcast(jnp.int32(0x5F3759DF) - lax.shift_right_logical(bits, 1),
                         jnp.float32)
        for _ in range(3):
            y = y * (1.5 - 0.5 * acc * y * y)
        ov[pl.ds(blk * LANES, LANES)] = acc * y
        return carry

    lax.fori_loop(0, BPW // LANES, block, 0)
    pltpu.sync_copy(ov, out_hbm.at[pl.ds(base, BPW)])


_sc_call = pl.kernel(
    _tec_body,
    out_type=jax.ShapeDtypeStruct((BATCH_SIZE,), jnp.float32),
    mesh=plsc.VectorSubcoreMesh(core_axis_name="c", subcore_axis_name="s"),
    compiler_params=pltpu.CompilerParams(needs_layout_passes=False,
                                         use_tc_tiling_on_sc=False),
    scratch_types=[
        pltpu.VMEM((NCHUNK, CHUNK), jnp.int32),
        pltpu.VMEM((NCHUNK, CHUNK), jnp.int32),
        pltpu.VMEM((NCHUNK, CHUNK), jnp.int32),
        pltpu.VMEM((BPW, DIM), jnp.float32),
        pltpu.VMEM((BPW, DIM), jnp.float32),
        pltpu.VMEM((BPW, DIM), jnp.float32),
        pltpu.VMEM((BPW,), jnp.float32),
        pltpu.SemaphoreType.DMA,
    ],
)


def kernel(h, r, t, entity_embedding, relation_embedding):
    h3 = h.astype(jnp.int32).reshape(NW, NCHUNK, CHUNK)
    r3 = r.astype(jnp.int32).reshape(NW, NCHUNK, CHUNK)
    t3 = t.astype(jnp.int32).reshape(NW, NCHUNK, CHUNK)
    return _sc_call(h3, r3, t3, entity_embedding, relation_embedding)


# speedup vs baseline: 1.0681x; 1.0681x over previous
import jax
import jax.numpy as jnp
from jax import lax
from jax.experimental import pallas as pl
from jax.experimental.pallas import tpu as pltpu
from jax.experimental.pallas import tpu_sc as plsc

NUM_CORES = 2
NUM_SUBCORES = 16
LANES = 16
NW = NUM_CORES * NUM_SUBCORES
BATCH_SIZE = 16384
DIM = 64
BPW = BATCH_SIZE // NW
CHUNK = 128
NCHUNK = BPW // CHUNK
BLOCKS_PER_CHUNK = CHUNK // LANES


def _tec_body(h_hbm, r_hbm, t_hbm, ent_hbm, rel_hbm, out_hbm,
              hi, ri, ti, hv, rv, tv, ov, ps, sem):
    wid = lax.axis_index("s") * NUM_CORES + lax.axis_index("c")
    base = wid * BPW

    pltpu.sync_copy(h_hbm.at[wid], hi)
    pltpu.sync_copy(r_hbm.at[wid], ri)
    pltpu.sync_copy(t_hbm.at[wid], ti)

    for c in range(NCHUNK):
        dst = pl.ds(c * CHUNK, CHUNK)
        pltpu.async_copy(ent_hbm.at[hi.at[c]], hv.at[dst], sem)
        pltpu.async_copy(ent_hbm.at[ti.at[c]], tv.at[dst], sem)
        pltpu.async_copy(rel_hbm.at[ri.at[c]], rv.at[dst], sem)

    iota16 = lax.iota(jnp.int32, LANES)
    rowbase = iota16 * LANES

    def block(blk, carry):
        for j in range(LANES):
            i = blk * LANES + j
            sq = []
            for a in range(4):
                sl = pl.ds(a * LANES, LANES)
                df = (hv.at[i][sl] - tv.at[i][sl]) + rv.at[i][sl]
                sq.append(df * df)
            ps[pl.ds(j * LANES, LANES)] = (sq[0] + sq[1]) + (sq[2] + sq[3])

        acc = plsc.load_gather(ps, [rowbase])
        for l in range(1, LANES):
            acc = acc + plsc.load_gather(ps, [rowbase + l])

        bits = plsc.bitcast(acc, jnp.int32)
        y = plsc.bitcast(jnp.int32(0x5F3759DF) - lax.shift_right_logical(bits, 1),
                         jnp.float32)
        for _ in range(3):
            y = y * (1.5 - 0.5 * acc * y * y)
        ov[pl.ds(blk * LANES, LANES)] = acc * y
        return carry

    for c in range(NCHUNK):
        dst = pl.ds(c * CHUNK, CHUNK)
        pltpu.make_async_copy(ent_hbm.at[hi.at[c]], hv.at[dst], sem).wait()
        pltpu.make_async_copy(ent_hbm.at[ti.at[c]], tv.at[dst], sem).wait()
        pltpu.make_async_copy(rel_hbm.at[ri.at[c]], rv.at[dst], sem).wait()
        lax.fori_loop(c * BLOCKS_PER_CHUNK, (c + 1) * BLOCKS_PER_CHUNK,
                      block, 0)

    pltpu.sync_copy(ov, out_hbm.at[pl.ds(base, BPW)])


_sc_call = pl.kernel(
    _tec_body,
    out_type=jax.ShapeDtypeStruct((BATCH_SIZE,), jnp.float32),
    mesh=plsc.VectorSubcoreMesh(core_axis_name="c", subcore_axis_name="s"),
    compiler_params=pltpu.CompilerParams(needs_layout_passes=False,
                                         use_tc_tiling_on_sc=False),
    scratch_types=[
        pltpu.VMEM((NCHUNK, CHUNK), jnp.int32),
        pltpu.VMEM((NCHUNK, CHUNK), jnp.int32),
        pltpu.VMEM((NCHUNK, CHUNK), jnp.int32),
        pltpu.VMEM((BPW, DIM), jnp.float32),
        pltpu.VMEM((BPW, DIM), jnp.float32),
        pltpu.VMEM((BPW, DIM), jnp.float32),
        pltpu.VMEM((BPW,), jnp.float32),
        pltpu.VMEM((LANES * LANES,), jnp.float32),
        pltpu.SemaphoreType.DMA,
    ],
)


def kernel(h, r, t, entity_embedding, relation_embedding):
    h3 = h.astype(jnp.int32).reshape(NW, NCHUNK, CHUNK)
    r3 = r.astype(jnp.int32).reshape(NW, NCHUNK, CHUNK)
    t3 = t.astype(jnp.int32).reshape(NW, NCHUNK, CHUNK)
    return _sc_call(h3, r3, t3, entity_embedding, relation_embedding)
